# Initial kernel scaffold; baseline (speedup 1.0000x reference)
import functools

import jax
import jax.numpy as jnp
from jax import lax
from jax.experimental import pallas as pl
from jax.experimental.pallas import tpu as pltpu
from jax.experimental.pallas import tpu_sc as plsc

_NUM_SAMPLES = 10000
_K = 80


def _sc_partial_segment_sum(emb, ids2d, zeros, *, n, d, s):
    nc, ns = 2, 16
    nw = nc * ns
    cn = n // nw
    iters = cn // _K
    gs = s // ns
    mesh = plsc.VectorSubcoreMesh(core_axis_name="c", subcore_axis_name="s")

    @functools.partial(
        pl.kernel,
        out_type=jax.ShapeDtypeStruct((nc, s, d), jnp.float32),
        mesh=mesh,
        scratch_types=[
            pltpu.VMEM((iters, _K), jnp.int32),
            pltpu.VMEM((_K, d), jnp.float32),
            pltpu.VMEM_SHARED((s, d), jnp.float32),
        ],
    )
    def k(emb_hbm, ids_hbm, zeros_hbm, out_hbm, idx_v, rows_v, acc):
        c = lax.axis_index("c")
        sub = lax.axis_index("s")
        wid = c * ns + sub

        pltpu.sync_copy(zeros_hbm, acc.at[pl.ds(sub * gs, gs)])
        pltpu.sync_copy(ids_hbm.at[pl.ds(wid * iters, iters)], idx_v)
        plsc.subcore_barrier()

        def step(i, carry):
            base = wid * cn + i * _K
            pltpu.sync_copy(emb_hbm.at[pl.ds(base, _K)], rows_v)
            pltpu.sync_copy(rows_v, acc.at[idx_v.at[i]], add=True)
            return carry

        lax.fori_loop(0, iters, step, 0)
        plsc.subcore_barrier()
        pltpu.sync_copy(acc.at[pl.ds(sub * gs, gs)],
                        out_hbm.at[c, pl.ds(sub * gs, gs)])

    return k(emb, ids2d, zeros)


def _merge_body(p_ref, o_ref):
    o_ref[...] = p_ref[0] + p_ref[1]


def kernel(element_embeddings, element_to_sample_map, num_samples):
    n, d = element_embeddings.shape
    s = _NUM_SAMPLES
    ids = jnp.clip(element_to_sample_map.astype(jnp.int32), 0, s - 1)
    ids2d = ids.reshape(n // _K, _K)
    zeros = jnp.zeros((s // 16, d), jnp.float32)
    partials = _sc_partial_segment_sum(element_embeddings, ids2d, zeros,
                                       n=n, d=d, s=s)
    blk = s // 8
    return pl.pallas_call(
        _merge_body,
        out_shape=jax.ShapeDtypeStruct((s, d), jnp.float32),
        grid=(8,),
        in_specs=[pl.BlockSpec((2, blk, d), lambda i: (0, i, 0))],
        out_specs=pl.BlockSpec((blk, d), lambda i: (i, 0)),
    )(partials)


# speedup vs baseline: 4.4340x; 4.4340x over previous
import functools

import jax
import jax.numpy as jnp
from jax import lax
from jax.experimental import pallas as pl
from jax.experimental.pallas import tpu as pltpu
from jax.experimental.pallas import tpu_sc as plsc

_NUM_SAMPLES = 10000
_K = 80


def _sc_partial_segment_sum(emb, ids3d, zeros, *, n, d, sp):
    nc, ns = 2, 16
    nw = nc * ns
    cn = n // nw
    iters = cn // _K
    gs = sp // ns
    mesh = plsc.VectorSubcoreMesh(core_axis_name="c", subcore_axis_name="s")

    @functools.partial(
        pl.kernel,
        out_type=jax.ShapeDtypeStruct((nc, sp, d), jnp.float32),
        mesh=mesh,
        scratch_types=[
            pltpu.VMEM((iters, _K), jnp.int32),
            pltpu.VMEM((_K, d), jnp.float32),
            pltpu.VMEM_SHARED((sp, d), jnp.float32),
        ],
    )
    def k(emb_hbm, ids_hbm, zeros_hbm, out_hbm, idx_v, rows_v, acc):
        c = lax.axis_index("c")
        sub = lax.axis_index("s")
        wid = c * ns + sub

        pltpu.sync_copy(zeros_hbm, acc.at[pl.ds(sub * gs, gs)])
        pltpu.sync_copy(ids_hbm.at[wid], idx_v)
        plsc.subcore_barrier()

        def step(i, carry):
            base = wid * cn + i * _K
            pltpu.sync_copy(emb_hbm.at[pl.ds(base, _K)], rows_v)
            pltpu.sync_copy(rows_v, acc.at[idx_v.at[i]], add=True)
            return carry

        lax.fori_loop(0, iters, step, 0)
        plsc.subcore_barrier()
        pltpu.sync_copy(acc.at[pl.ds(sub * gs, gs)],
                        out_hbm.at[c, pl.ds(sub * gs, gs)])

    return k(emb, ids3d, zeros)


def _merge_body(p_ref, o_ref):
    o_ref[...] = p_ref[0] + p_ref[1]


def kernel(element_embeddings, element_to_sample_map, num_samples):
    n, d = element_embeddings.shape
    s = _NUM_SAMPLES
    sp = 10240
    nw = 32
    ids = jnp.clip(element_to_sample_map.astype(jnp.int32), 0, s - 1)
    ids3d = ids.reshape(nw, (n // nw) // _K, _K)
    zeros = jnp.zeros((sp // 16, d), jnp.float32)
    partials = _sc_partial_segment_sum(element_embeddings, ids3d, zeros,
                                       n=n, d=d, sp=sp)
    blk = sp // 10
    merged = pl.pallas_call(
        _merge_body,
        out_shape=jax.ShapeDtypeStruct((sp, d), jnp.float32),
        grid=(10,),
        in_specs=[pl.BlockSpec((2, blk, d), lambda i: (0, i, 0))],
        out_specs=pl.BlockSpec((blk, d), lambda i: (i, 0)),
    )(partials)
    return merged[:s]


# speedup vs baseline: 6.8803x; 1.5517x over previous
import functools

import jax
import jax.numpy as jnp
from jax import lax
from jax.experimental import pallas as pl
from jax.experimental.pallas import tpu as pltpu
from jax.experimental.pallas import tpu_sc as plsc

_NUM_SAMPLES = 10000
_K = 80


def _sc_partial_segment_sum(emb, ids3d, zeros, *, n, d, sp):
    nc, ns = 2, 16
    nw = nc * ns
    cn = n // nw
    iters = cn // _K
    gs = sp // ns
    mesh = plsc.VectorSubcoreMesh(core_axis_name="c", subcore_axis_name="s")

    spb = 1
    b_rows = spb * _K
    nblk = cn // b_rows
    npair = (nblk - 1) // 2

    @functools.partial(
        pl.kernel,
        out_type=jax.ShapeDtypeStruct((nc, sp, d), jnp.float32),
        mesh=mesh,
        scratch_types=[
            pltpu.VMEM((iters, _K), jnp.int32),
            pltpu.VMEM((b_rows, d), jnp.float32),
            pltpu.VMEM((b_rows, d), jnp.float32),
            pltpu.VMEM_SHARED((sp, d), jnp.float32),
            pltpu.SemaphoreType.DMA,
            pltpu.SemaphoreType.DMA,
            pltpu.SemaphoreType.DMA,
            pltpu.SemaphoreType.DMA,
        ],
    )
    def k(emb_hbm, ids_hbm, zeros_hbm, out_hbm,
          idx_v, rows0, rows1, acc, ls0, ls1, ss0, ss1):
        c = lax.axis_index("c")
        sub = lax.axis_index("s")
        wid = c * ns + sub

        def load_desc(buf, sem, blk):
            src = emb_hbm.at[pl.ds(wid * cn + blk * b_rows, b_rows)]
            return pltpu.make_async_copy(src, buf, sem)

        def scat_start(buf, sem, blk):
            for j in range(spb):
                pltpu.async_copy(buf.at[pl.ds(j * _K, _K)],
                                 acc.at[idx_v.at[blk * spb + j]], sem, add=True)

        def scat_wait(buf, sem, blk):
            for j in range(spb):
                pltpu.make_async_copy(buf.at[pl.ds(j * _K, _K)],
                                      acc.at[idx_v.at[blk * spb + j]], sem).wait()

        pltpu.sync_copy(zeros_hbm, acc.at[pl.ds(sub * gs, gs)])
        pltpu.sync_copy(ids_hbm.at[wid], idx_v)
        plsc.subcore_barrier()

        load_desc(rows0, ls0, 0).start()

        def pair(t, carry):
            b1 = 2 * t + 1

            @pl.when(t > 0)
            def _():
                scat_wait(rows1, ss1, b1 - 2)

            load_desc(rows1, ls1, b1).start()
            load_desc(rows0, ls0, b1 - 1).wait()
            scat_start(rows0, ss0, b1 - 1)
            scat_wait(rows0, ss0, b1 - 1)
            load_desc(rows0, ls0, b1 + 1).start()
            load_desc(rows1, ls1, b1).wait()
            scat_start(rows1, ss1, b1)
            return carry

        lax.fori_loop(0, npair, pair, 0)
        last = 2 * npair
        scat_wait(rows1, ss1, last - 1)
        load_desc(rows0, ls0, last).wait()
        scat_start(rows0, ss0, last)
        scat_wait(rows0, ss0, last)

        plsc.subcore_barrier()
        pltpu.sync_copy(acc.at[pl.ds(sub * gs, gs)],
                        out_hbm.at[c, pl.ds(sub * gs, gs)])

    return k(emb, ids3d, zeros)


def _merge_body(p_ref, o_ref):
    o_ref[...] = p_ref[0] + p_ref[1]


def kernel(element_embeddings, element_to_sample_map, num_samples):
    n, d = element_embeddings.shape
    s = _NUM_SAMPLES
    sp = 10240
    nw = 32
    ids = jnp.clip(element_to_sample_map.astype(jnp.int32), 0, s - 1)
    ids3d = ids.reshape(nw, (n // nw) // _K, _K)
    zeros = jnp.zeros((sp // 16, d), jnp.float32)
    partials = _sc_partial_segment_sum(element_embeddings, ids3d, zeros,
                                       n=n, d=d, sp=sp)
    blk = sp // 10
    merged = pl.pallas_call(
        _merge_body,
        out_shape=jax.ShapeDtypeStruct((sp, d), jnp.float32),
        grid=(10,),
        in_specs=[pl.BlockSpec((2, blk, d), lambda i: (0, i, 0))],
        out_specs=pl.BlockSpec((blk, d), lambda i: (i, 0)),
    )(partials)
    return merged[:s]


# speedup vs baseline: 7.4040x; 1.0761x over previous
import functools

import jax
import jax.numpy as jnp
from jax import lax
from jax.experimental import pallas as pl
from jax.experimental.pallas import tpu as pltpu
from jax.experimental.pallas import tpu_sc as plsc

_NUM_SAMPLES = 10000
_K = 80


def _sc_partial_segment_sum(emb, ids, zeros, *, n, d, sp):
    nc, ns = 2, 16
    nw = nc * ns
    cn = n // nw
    nblk = cn // _K
    gs = sp // ns
    nbuf = 4
    mesh = plsc.VectorSubcoreMesh(core_axis_name="c", subcore_axis_name="s")

    @functools.partial(
        pl.kernel,
        out_type=jax.ShapeDtypeStruct((nc, sp, d), jnp.float32),
        mesh=mesh,
        scratch_types=[
            [pltpu.VMEM((_K, d), jnp.float32) for _ in range(nbuf)],
            [pltpu.VMEM((_K,), jnp.int32) for _ in range(nbuf)],
            pltpu.VMEM_SHARED((sp, d), jnp.float32),
            [pltpu.SemaphoreType.DMA for _ in range(nbuf)],
            [pltpu.SemaphoreType.DMA for _ in range(nbuf)],
            [pltpu.SemaphoreType.DMA for _ in range(nbuf)],
        ],
    )
    def k(emb_hbm, ids_hbm, zeros_hbm, out_hbm,
          rows, idxs, acc, lsem, isem, ssem):
        c = lax.axis_index("c")
        sub = lax.axis_index("s")
        wid = c * ns + sub

        def row_desc(j, blk):
            src = emb_hbm.at[pl.ds(wid * cn + blk * _K, _K)]
            return pltpu.make_async_copy(src, rows[j], lsem[j])

        def idx_desc(j, blk):
            src = ids_hbm.at[pl.ds(wid * cn + blk * _K, _K)]
            return pltpu.make_async_copy(src, idxs[j], isem[j])

        def load_start(j, blk):
            row_desc(j, blk).start()
            idx_desc(j, blk).start()

        def load_wait(j, blk):
            row_desc(j, blk).wait()
            idx_desc(j, blk).wait()

        def scat_start(j):
            pltpu.async_copy(rows[j], acc.at[idxs[j]], ssem[j], add=True)

        def scat_wait(j):
            pltpu.make_async_copy(rows[j], acc.at[idxs[j]], ssem[j]).wait()

        pltpu.sync_copy(zeros_hbm, acc.at[pl.ds(sub * gs, gs)])
        plsc.subcore_barrier()

        load_start(0, 0)
        load_start(1, 1)

        def group(g, carry):
            for j in range(nbuf):
                i = nbuf * g + j
                load_wait(j, i)
                scat_start(j)
                jj = (j + 2) % nbuf

                @pl.when(i >= 2)
                def _():
                    scat_wait(jj)

                @pl.when(i + 2 <= nblk - 1)
                def _():
                    load_start(jj, i + 2)
            return carry

        lax.fori_loop(0, (nblk - 1) // nbuf, group, 0)
        last = nblk - 1
        load_wait(last % nbuf, last)
        scat_start(last % nbuf)
        scat_wait((last - 2) % nbuf)
        scat_wait((last - 1) % nbuf)
        scat_wait(last % nbuf)

        plsc.subcore_barrier()
        pltpu.sync_copy(acc.at[pl.ds(sub * gs, gs)],
                        out_hbm.at[c, pl.ds(sub * gs, gs)])

    return k(emb, ids, zeros)


def _merge_body(p_ref, o_ref):
    o_ref[...] = p_ref[0] + p_ref[1]


def kernel(element_embeddings, element_to_sample_map, num_samples):
    n, d = element_embeddings.shape
    s = _NUM_SAMPLES
    sp = 10240
    ids = jnp.clip(element_to_sample_map.astype(jnp.int32), 0, s - 1)
    zeros = jnp.zeros((sp // 16, d), jnp.float32)
    partials = _sc_partial_segment_sum(element_embeddings, ids, zeros,
                                       n=n, d=d, sp=sp)
    blk = s // 10
    return pl.pallas_call(
        _merge_body,
        out_shape=jax.ShapeDtypeStruct((s, d), jnp.float32),
        grid=(10,),
        in_specs=[pl.BlockSpec((2, blk, d), lambda i: (0, i, 0))],
        out_specs=pl.BlockSpec((blk, d), lambda i: (i, 0)),
    )(partials)
